# baseline (speedup 1.0000x reference)
import functools

import jax
import jax.numpy as jnp
from jax import lax
from jax.experimental import pallas as pl
from jax.experimental.pallas import tpu as pltpu
from jax.experimental.pallas import tpu_sc as plsc


def _tc_pack_quarters(table, rows_per_blk=1000):
    V, D = table.shape
    V4 = V // 4
    nblk = V4 // rows_per_blk

    def pack(x0, x1, x2, x3, o_ref):
        o_ref[:, 0:D] = x0[...]
        o_ref[:, D:2 * D] = x1[...]
        o_ref[:, 2 * D:3 * D] = x2[...]
        o_ref[:, 3 * D:4 * D] = x3[...]

    def spec(k):
        return pl.BlockSpec((rows_per_blk, D), lambda i, k=k: (i + k * nblk, 0))

    return pl.pallas_call(
        pack,
        grid=(nblk,),
        in_specs=[spec(0), spec(1), spec(2), spec(3)],
        out_specs=pl.BlockSpec((rows_per_blk, 4 * D), lambda i: (i, 0)),
        out_shape=jax.ShapeDtypeStruct((V4, 4 * D), jnp.float32),
    )(table, table, table, table)


def _sc_gather_packed(table_packed, idxq, T, DP):
    info = plsc.get_sparse_core_info()
    NW = info.num_cores * info.num_subcores
    b_per_w = T // NW

    mesh = plsc.VectorSubcoreMesh(core_axis_name="c", subcore_axis_name="s")

    @functools.partial(
        pl.kernel,
        mesh=mesh,
        out_type=jax.ShapeDtypeStruct((T, DP), jnp.float32),
        scratch_types=[
            pltpu.VMEM((b_per_w,), jnp.int32),
            pltpu.VMEM((b_per_w, DP), jnp.float32),
            pltpu.SemaphoreType.DMA,
        ],
    )
    def gather_kernel(table_hbm, idx_hbm, out_hbm, idx_v, rows_v, sem):
        wid = lax.axis_index("s") * info.num_cores + lax.axis_index("c")
        base = wid * b_per_w
        pltpu.sync_copy(idx_hbm.at[pl.ds(base, b_per_w)], idx_v)
        pltpu.async_copy(table_hbm.at[idx_v], rows_v, sem).wait()
        pltpu.sync_copy(rows_v, out_hbm.at[pl.ds(base, b_per_w)])

    return gather_kernel(table_packed, idxq)


def _tc_select_project(xp, sel, W, b, vt=2048):
    T = xp.shape[0]
    V, D = W.shape
    nv = pl.cdiv(V, vt)

    def mm(xp_ref, sel_ref, w_ref, b_ref, o_ref):
        s = sel_ref[...]
        x = jnp.where(s == 0, xp_ref[:, 0:D], 0.0)
        x += jnp.where(s == 1, xp_ref[:, D:2 * D], 0.0)
        x += jnp.where(s == 2, xp_ref[:, 2 * D:3 * D], 0.0)
        x += jnp.where(s == 3, xp_ref[:, 3 * D:4 * D], 0.0)
        o_ref[...] = lax.dot_general(
            x, w_ref[...],
            (((1,), (1,)), ((), ())),
            preferred_element_type=jnp.float32,
        ) + b_ref[...]

    return pl.pallas_call(
        mm,
        grid=(nv,),
        in_specs=[
            pl.BlockSpec((T, 4 * D), lambda j: (0, 0)),
            pl.BlockSpec((T, 1), lambda j: (0, 0)),
            pl.BlockSpec((vt, D), lambda j: (j, 0)),
            pl.BlockSpec((1, vt), lambda j: (0, j)),
        ],
        out_specs=pl.BlockSpec((T, vt), lambda j: (0, j)),
        out_shape=jax.ShapeDtypeStruct((T, V), jnp.float32),
        compiler_params=pltpu.CompilerParams(
            dimension_semantics=("arbitrary",),
        ),
    )(xp, sel, W, b.reshape(1, V))


def kernel(input_ids, embed_table, W_proj, b_proj):
    B, T = input_ids.shape
    V, D = embed_table.shape
    ids = input_ids.reshape(T).astype(jnp.int32)
    table_packed = _tc_pack_quarters(embed_table)
    xp = _sc_gather_packed(table_packed, ids % (V // 4), T, 4 * D)
    logits = _tc_select_project(xp, (ids // (V // 4)).reshape(T, 1),
                                W_proj, b_proj)
    return logits.reshape(B, T, V)


# speedup vs baseline: 2.1879x; 2.1879x over previous
import functools

import jax
import jax.numpy as jnp
from jax import lax
from jax.experimental import pallas as pl
from jax.experimental.pallas import tpu as pltpu
from jax.experimental.pallas import tpu_sc as plsc


def _sc_gather(table, idx, T, D):
    info = plsc.get_sparse_core_info()
    NW = info.num_cores * info.num_subcores
    b_per_w = T // NW

    mesh = plsc.VectorSubcoreMesh(core_axis_name="c", subcore_axis_name="s")

    @functools.partial(
        pl.kernel,
        mesh=mesh,
        out_type=jax.ShapeDtypeStruct((T, D), jnp.float32),
        scratch_types=[
            pltpu.VMEM((b_per_w,), jnp.int32),
            pltpu.VMEM((b_per_w, D), jnp.float32),
            pltpu.SemaphoreType.DMA,
        ],
        compiler_params=pltpu.CompilerParams(use_tc_tiling_on_sc=False),
    )
    def gather_kernel(table_hbm, idx_hbm, out_hbm, idx_v, rows_v, sem):
        wid = lax.axis_index("s") * info.num_cores + lax.axis_index("c")
        base = wid * b_per_w
        pltpu.sync_copy(idx_hbm.at[pl.ds(base, b_per_w)], idx_v)
        pltpu.async_copy(table_hbm.at[idx_v], rows_v, sem).wait()
        pltpu.sync_copy(rows_v, out_hbm.at[pl.ds(base, b_per_w)])

    return gather_kernel(table, idx)


def _tc_project_t(x, W, b, vt=1024):
    T, D = x.shape
    V = W.shape[0]
    nv = pl.cdiv(V, vt)

    def mm(w_ref, x_ref, b_ref, o_ref):
        o_ref[...] = lax.dot_general(
            w_ref[...], x_ref[...],
            (((1,), (1,)), ((), ())),
            preferred_element_type=jnp.float32,
        ) + b_ref[...]

    return pl.pallas_call(
        mm,
        grid=(nv,),
        in_specs=[
            pl.BlockSpec((vt, D), lambda j: (j, 0)),
            pl.BlockSpec((T, D), lambda j: (0, 0)),
            pl.BlockSpec((vt, 1), lambda j: (j, 0)),
        ],
        out_specs=pl.BlockSpec((vt, T), lambda j: (j, 0)),
        out_shape=jax.ShapeDtypeStruct((V, T), jnp.float32),
        compiler_params=pltpu.CompilerParams(
            dimension_semantics=("arbitrary",),
        ),
    )(W, x, b.reshape(V, 1))


def kernel(input_ids, embed_table, W_proj, b_proj):
    B, T = input_ids.shape
    V, D = embed_table.shape
    ids = input_ids.reshape(T).astype(jnp.int32)
    x = _sc_gather(embed_table, ids, T, D)
    logits_t = _tc_project_t(x, W_proj, b_proj)
    return logits_t.T.reshape(B, T, V)


# speedup vs baseline: 2.2472x; 1.0271x over previous
import functools

import jax
import jax.numpy as jnp
from jax import lax
from jax.experimental import pallas as pl
from jax.experimental.pallas import tpu as pltpu
from jax.experimental.pallas import tpu_sc as plsc


def _sc_gather(table, idx, T, D):
    info = plsc.get_sparse_core_info()
    NW = info.num_cores * info.num_subcores
    b_per_w = T // NW

    mesh = plsc.VectorSubcoreMesh(core_axis_name="c", subcore_axis_name="s")

    @functools.partial(
        pl.kernel,
        mesh=mesh,
        out_type=jax.ShapeDtypeStruct((T, D), jnp.float32),
        scratch_types=[
            pltpu.VMEM((b_per_w,), jnp.int32),
            pltpu.VMEM((b_per_w, D), jnp.float32),
            pltpu.SemaphoreType.DMA,
        ],
        compiler_params=pltpu.CompilerParams(use_tc_tiling_on_sc=False),
    )
    def gather_kernel(table_hbm, idx_hbm, out_hbm, idx_v, rows_v, sem):
        wid = lax.axis_index("s") * info.num_cores + lax.axis_index("c")
        base = wid * b_per_w
        pltpu.sync_copy(idx_hbm.at[pl.ds(base, b_per_w)], idx_v)
        pltpu.async_copy(table_hbm.at[idx_v], rows_v, sem).wait()
        pltpu.sync_copy(rows_v, out_hbm.at[pl.ds(base, b_per_w)])

    return gather_kernel(table, idx)


def _tc_project_t(x, W, b, vt=1024):
    T, D = x.shape
    V = W.shape[0]
    nv = pl.cdiv(V, vt)

    def mm(w_ref, x_ref, b_ref, o_ref):
        o_ref[...] = lax.dot_general(
            w_ref[...], x_ref[...],
            (((1,), (1,)), ((), ())),
            preferred_element_type=jnp.float32,
        ) + b_ref[...]

    return pl.pallas_call(
        mm,
        grid=(nv,),
        in_specs=[
            pl.BlockSpec((vt, D), lambda j: (j, 0)),
            pl.BlockSpec((T, D), lambda j: (0, 0)),
            pl.BlockSpec((vt, 1), lambda j: (j, 0)),
        ],
        out_specs=pl.BlockSpec((vt, T), lambda j: (j, 0)),
        out_shape=jax.ShapeDtypeStruct((V, T), jnp.float32),
        compiler_params=pltpu.CompilerParams(
            dimension_semantics=("arbitrary",),
        ),
    )(W.astype(jnp.bfloat16), x.astype(jnp.bfloat16), b.reshape(V, 1))


def kernel(input_ids, embed_table, W_proj, b_proj):
    B, T = input_ids.shape
    V, D = embed_table.shape
    ids = input_ids.reshape(T).astype(jnp.int32)
    x = _sc_gather(embed_table, ids, T, D)
    logits_t = _tc_project_t(x, W_proj, b_proj)
    return logits_t.T.reshape(B, T, V)


# speedup vs baseline: 2.2671x; 1.0089x over previous
import functools

import jax
import jax.numpy as jnp
from jax import lax
from jax.experimental import pallas as pl
from jax.experimental.pallas import tpu as pltpu
from jax.experimental.pallas import tpu_sc as plsc


def _sc_gather(table, idx, T, D):
    info = plsc.get_sparse_core_info()
    NW = info.num_cores * info.num_subcores
    b_per_w = T // NW

    mesh = plsc.VectorSubcoreMesh(core_axis_name="c", subcore_axis_name="s")

    @functools.partial(
        pl.kernel,
        mesh=mesh,
        out_type=jax.ShapeDtypeStruct((T, D), jnp.float32),
        scratch_types=[
            pltpu.VMEM((b_per_w,), jnp.int32),
            pltpu.VMEM((b_per_w, D), jnp.float32),
            pltpu.SemaphoreType.DMA,
        ],
        compiler_params=pltpu.CompilerParams(use_tc_tiling_on_sc=False),
    )
    def gather_kernel(table_hbm, idx_hbm, out_hbm, idx_v, rows_v, sem):
        wid = lax.axis_index("s") * info.num_cores + lax.axis_index("c")
        base = wid * b_per_w
        pltpu.sync_copy(idx_hbm.at[pl.ds(base, b_per_w)], idx_v)
        pltpu.async_copy(table_hbm.at[idx_v], rows_v, sem).wait()
        pltpu.sync_copy(rows_v, out_hbm.at[pl.ds(base, b_per_w)])

    return gather_kernel(table, idx)


def _tc_project_t(x, W, b, vt=1600):
    T, D = x.shape
    V = W.shape[0]
    nv = pl.cdiv(V, vt)

    def mm(w_ref, x_ref, b_ref, o_ref):
        o_ref[...] = lax.dot_general(
            w_ref[...], x_ref[...],
            (((1,), (1,)), ((), ())),
            preferred_element_type=jnp.float32,
        ) + b_ref[...]

    return pl.pallas_call(
        mm,
        grid=(nv,),
        in_specs=[
            pl.BlockSpec((vt, D), lambda j: (j, 0)),
            pl.BlockSpec((T, D), lambda j: (0, 0)),
            pl.BlockSpec((vt, 1), lambda j: (j, 0)),
        ],
        out_specs=pl.BlockSpec((vt, T), lambda j: (j, 0)),
        out_shape=jax.ShapeDtypeStruct((V, T), jnp.float32),
        compiler_params=pltpu.CompilerParams(
            dimension_semantics=("arbitrary",),
        ),
    )(W.astype(jnp.bfloat16), x.astype(jnp.bfloat16), b.reshape(V, 1))


def kernel(input_ids, embed_table, W_proj, b_proj):
    B, T = input_ids.shape
    V, D = embed_table.shape
    ids = input_ids.reshape(T).astype(jnp.int32)
    x = _sc_gather(embed_table, ids, T, D)
    logits_t = _tc_project_t(x, W_proj, b_proj)
    return logits_t.T.reshape(B, T, V)
